# baseline (speedup 1.0000x reference)
import functools
import jax
import jax.numpy as jnp
from jax import lax
from jax.experimental import pallas as pl
from jax.experimental.pallas import tpu as pltpu
from jax.experimental.pallas import tpu_sc as plsc

_N = 10000
_E = 320000
_D = 128
_C = 40
_G = 64

_NC = 2
_NS = 16
_NW = _NC * _NS
_L = 16

_NP = 10240
_SLC = _NP // _NS
_GN = _G * _NP
_RSP = _GN // _NS
_ZCH = 10240

_EW = 9984
_K = 1664
_EV = 2 * _EW
_DK = 3328
_EX = 512
_XOFF = _NW * _EW


def _rsqrt16(deg):
    magic = jnp.full((_L,), 0x5F3759DF, jnp.int32)
    half = deg * jnp.full((_L,), 0.5, jnp.float32)
    three_half = jnp.full((_L,), 1.5, jnp.float32)
    y = plsc.bitcast(magic - (plsc.bitcast(deg, jnp.int32) >> 1), jnp.float32)
    for _ in range(3):
        y = y * (three_half - half * y * y)
    return y


def _sc_body(ei_hbm, batch_hbm, r_hbm, cnt_hbm,
             eb_v, ebx_v, batch_v, dis_v, disl_v, z_v, dval_v,
             dbuf0, dbuf1, didx0, didx1, rval0, rval1, ridx0, ridx1,
             rvx_v, rix_v, val_s, idx_s,
             s_edge, s_ebx, s_batch, s_dl0, s_dl1, s_ds0, s_ds1,
             s_rs0, s_rs1, s_o0, s_o1,
             sh_deg, sh_dis, sh_r):
    cid = lax.axis_index("c")
    sid = lax.axis_index("s")
    wid = cid * _NS + sid

    zeros16 = jnp.zeros((_L,), jnp.float32)
    ones16 = jnp.full((_L,), 1.0, jnp.float32)

    dbuf = (dbuf0, dbuf1)
    didx = (didx0, didx1)
    s_dl = (s_dl0, s_dl1)
    s_ds = (s_ds0, s_ds1)
    rval = (rval0, rval1)
    ridx = (ridx0, ridx1)
    s_rs = (s_rs0, s_rs1)
    outb = (z_v, dis_v)
    s_o = (s_o0, s_o1)
    n_deg = _EV // _DK
    n_r = _EW // _K

    c_edge = pltpu.async_copy(ei_hbm.at[:, pl.ds(wid * _EW, _EW)],
                              eb_v, s_edge)
    c_bat = pltpu.async_copy(batch_hbm, batch_v.at[pl.ds(0, _N)], s_batch)

    @pl.when(sid == _NS - 1)
    def _():
        pltpu.async_copy(ei_hbm.at[:, pl.ds(_XOFF, _EX)], ebx_v, s_ebx)
    dload = [None] * n_deg
    for j in range(2):
        dload[j] = pltpu.async_copy(
            ei_hbm.at[:, pl.ds(sid * _EV + j * _DK, _DK)], dbuf[j], s_dl[j])

    def _z2(i, _):
        z_v[pl.ds(i * _L, _L)] = zeros16
        return ()
    lax.fori_loop(0, _ZCH // _L, _z2, ())

    def _o(i, _):
        dval_v[pl.ds(i * _L, _L)] = ones16
        return ()
    lax.fori_loop(0, _DK // _L, _o, ())

    pltpu.sync_copy(z_v.at[pl.ds(0, _SLC)], sh_deg.at[pl.ds(sid * _SLC, _SLC)])
    for j in range(_RSP // _ZCH):
        pltpu.sync_copy(z_v, sh_r.at[pl.ds(sid * _RSP + j * _ZCH, _ZCH)])

    @pl.when(sid == 0)
    def _():
        pltpu.sync_copy(z_v.at[pl.ds(0, 128)], sh_r.at[pl.ds(_GN, 128)])
    plsc.subcore_barrier()

    dscat = [None] * n_deg
    for ci in range(n_deg):
        b = ci % 2
        dload[ci].wait()
        if ci >= 2:
            dscat[ci - 2].wait()

        def _ext(i, _, b=b):
            didx[b][pl.ds(i * _L, _L)] = dbuf[b][1, pl.ds(i * _L, _L)]
            return ()
        lax.fori_loop(0, _DK // _L, _ext, ())
        if ci + 2 < n_deg:
            dload[ci + 2] = pltpu.async_copy(
                ei_hbm.at[:, pl.ds(sid * _EV + (ci + 2) * _DK, _DK)],
                dbuf[b], s_dl[b])
        dscat[ci] = pltpu.async_copy(dval_v, sh_deg.at[didx[b]], s_ds[b],
                                     add=True)
    dscat[n_deg - 2].wait()
    dscat[n_deg - 1].wait()

    @pl.when(sid == _NS - 1)
    def _():
        pltpu.make_async_copy(ei_hbm.at[:, pl.ds(_XOFF, _EX)],
                              ebx_v, s_ebx).wait()

        def _extx(i, _):
            rix_v[pl.ds(i * _L, _L)] = ebx_v[1, pl.ds(i * _L, _L)]
            return ()
        lax.fori_loop(0, _EX // _L, _extx, ())
        pltpu.sync_copy(dval_v.at[pl.ds(0, _EX)], sh_deg.at[rix_v], add=True)
    plsc.subcore_barrier()

    pltpu.sync_copy(sh_deg.at[pl.ds(sid * _SLC, _SLC)], disl_v)

    def _red(k, _):
        acc = disl_v[pl.ds(k * _L, _L)]
        disv = _rsqrt16(acc + ones16)
        disl_v[pl.ds(k * _L, _L)] = disv
        return ()
    lax.fori_loop(0, _SLC // _L, _red, ())

    pltpu.sync_copy(disl_v, sh_dis.at[pl.ds(sid * _SLC, _SLC)])
    plsc.subcore_barrier()

    pltpu.sync_copy(sh_dis, dis_v)
    c_edge.wait()
    c_bat.wait()

    np16 = jnp.full((_L,), _NP, jnp.int32)

    @pl.when(cid == 0)
    def _():
        nlim16 = jnp.full((_L,), _N, jnp.int32)
        gn16 = jnp.full((_L,), _GN, jnp.int32)
        zeros16i = jnp.zeros((_L,), jnp.int32)

        def _selfloop(k, _):
            off = sid * _SLC + k * _L
            n16 = lax.iota(jnp.int32, _L) + jnp.full((_L,), 1, jnp.int32) * off
            dn = dis_v[pl.ds(off, _L)]
            gb = batch_v[pl.ds(off, _L)]
            valid = n16 < nlim16
            idx_s[pl.ds(2 * k * _L, _L)] = jnp.where(
                valid, gb * np16 + n16, zeros16i)
            val_s[pl.ds(2 * k * _L, _L)] = jnp.where(valid, dn * dn, zeros16)
            idx_s[pl.ds((2 * k + 1) * _L, _L)] = jnp.where(
                valid, gn16 + gb, zeros16i)
            val_s[pl.ds((2 * k + 1) * _L, _L)] = jnp.where(
                valid, ones16, zeros16)
            return ()
        lax.fori_loop(0, _SLC // _L, _selfloop, ())
        pltpu.sync_copy(val_s, sh_r.at[idx_s], add=True)

    rscat = [None] * n_r
    for ci in range(n_r):
        b = ci % 2
        if ci >= 2:
            rscat[ci - 2].wait()

        def _inner(i, _, ci=ci, b=b):
            off = ci * _K + i * _L
            s16 = eb_v[0, pl.ds(off, _L)]
            d16 = eb_v[1, pl.ds(off, _L)]
            vs = plsc.load_gather(dis_v, [s16])
            vd = plsc.load_gather(dis_v, [d16])
            gb = plsc.load_gather(batch_v, [d16])
            ridx[b][pl.ds(i * _L, _L)] = gb * np16 + s16
            rval[b][pl.ds(i * _L, _L)] = vs * vd
            return ()
        lax.fori_loop(0, _K // _L, _inner, ())
        rscat[ci] = pltpu.async_copy(rval[b], sh_r.at[ridx[b]], s_rs[b],
                                     add=True)
    rscat[n_r - 2].wait()
    rscat[n_r - 1].wait()

    @pl.when(wid == _NW - 1)
    def _():
        def _xinner(i, _):
            s16 = ebx_v[0, pl.ds(i * _L, _L)]
            d16 = ebx_v[1, pl.ds(i * _L, _L)]
            vs = plsc.load_gather(dis_v, [s16])
            vd = plsc.load_gather(dis_v, [d16])
            gb = plsc.load_gather(batch_v, [d16])
            rix_v[pl.ds(i * _L, _L)] = gb * np16 + s16
            rvx_v[pl.ds(i * _L, _L)] = vs * vd
            return ()
        lax.fori_loop(0, _EX // _L, _xinner, ())
        pltpu.sync_copy(rvx_v, sh_r.at[rix_v], add=True)
    plsc.subcore_barrier()

    ostore = [None, None]
    for j in range(_RSP // _ZCH):
        b = j % 2
        if ostore[b] is not None:
            ostore[b].wait()
        pltpu.sync_copy(sh_r.at[pl.ds(sid * _RSP + j * _ZCH, _ZCH)], outb[b])
        ostore[b] = pltpu.async_copy(
            outb[b],
            r_hbm.at[pl.ds(cid * _GN + sid * _RSP + j * _ZCH, _ZCH)], s_o[b])
    ostore[0].wait()
    ostore[1].wait()

    @pl.when((cid == 0) & (sid == 0))
    def _():
        pltpu.sync_copy(sh_r.at[pl.ds(_GN, 128)], disl_v.at[pl.ds(0, 128)])
        pltpu.sync_copy(disl_v.at[pl.ds(0, 128)], cnt_hbm)


def _sc_call(edge_index, batch):
    mesh = plsc.VectorSubcoreMesh(core_axis_name="c", subcore_axis_name="s",
                                  num_cores=_NC, num_subcores=_NS)
    return pl.kernel(
        _sc_body,
        out_type=(jax.ShapeDtypeStruct((_NC * _GN,), jnp.float32),
                  jax.ShapeDtypeStruct((128,), jnp.float32)),
        mesh=mesh,
        compiler_params=pltpu.CompilerParams(needs_layout_passes=False),
        scratch_types=[
            pltpu.VMEM((2, _EW), jnp.int32),
            pltpu.VMEM((2, _EX), jnp.int32),
            pltpu.VMEM((_NP,), jnp.int32),
            pltpu.VMEM((_NP,), jnp.float32),
            pltpu.VMEM((_SLC,), jnp.float32),
            pltpu.VMEM((_ZCH,), jnp.float32),
            pltpu.VMEM((_DK,), jnp.float32),
            pltpu.VMEM((2, _DK), jnp.int32),
            pltpu.VMEM((2, _DK), jnp.int32),
            pltpu.VMEM((_DK,), jnp.int32),
            pltpu.VMEM((_DK,), jnp.int32),
            pltpu.VMEM((_K,), jnp.float32),
            pltpu.VMEM((_K,), jnp.float32),
            pltpu.VMEM((_K,), jnp.int32),
            pltpu.VMEM((_K,), jnp.int32),
            pltpu.VMEM((_EX,), jnp.float32),
            pltpu.VMEM((_EX,), jnp.int32),
            pltpu.VMEM((2 * _SLC,), jnp.float32),
            pltpu.VMEM((2 * _SLC,), jnp.int32),
            pltpu.SemaphoreType.DMA,
            pltpu.SemaphoreType.DMA,
            pltpu.SemaphoreType.DMA,
            pltpu.SemaphoreType.DMA,
            pltpu.SemaphoreType.DMA,
            pltpu.SemaphoreType.DMA,
            pltpu.SemaphoreType.DMA,
            pltpu.SemaphoreType.DMA,
            pltpu.SemaphoreType.DMA,
            pltpu.SemaphoreType.DMA,
            pltpu.SemaphoreType.DMA,
            pltpu.VMEM_SHARED((_NP,), jnp.float32),
            pltpu.VMEM_SHARED((_NP,), jnp.float32),
            pltpu.VMEM_SHARED((_GN + 128,), jnp.float32),
        ],
    )(edge_index, batch)


_BLK = 1280
_NBLK = _NP // _BLK


def _tc_body(x_ref, w_ref, b_ref, rp_hbm, cnt_ref, out_ref, acc_ref, rp_v):
    i = pl.program_id(0)

    @pl.when(i == 0)
    def _():
        acc_ref[...] = jnp.zeros((_C, _NC * _G), jnp.float32)

    pltpu.sync_copy(rp_hbm.at[:, pl.ds(i * _BLK, _BLK)], rp_v)
    h = jnp.dot(x_ref[...], w_ref[...], preferred_element_type=jnp.float32)
    dn = (((0,), (1,)), ((), ()))
    acc_ref[...] += lax.dot_general(h, rp_v[...], dn,
                                    preferred_element_type=jnp.float32)

    @pl.when(i == _NBLK - 1)
    def _():
        acc = acc_ref[...]
        a64 = acc[:, :_G] + acc[:, _G:]
        cnt = cnt_ref[...][: _G].reshape(1, _G)
        pooled_t = a64 / jnp.maximum(cnt, 1.0)
        pooled_t = pooled_t + b_ref[...].T * jnp.where(cnt > 0.0, 1.0, 0.0)
        out_ref[...] = pooled_t.T


def _tc_call(x_pad, W, b_row, rp, cnt):
    return pl.pallas_call(
        _tc_body,
        grid=(_NBLK,),
        in_specs=[
            pl.BlockSpec((_BLK, _D), lambda i: (i, 0)),
            pl.BlockSpec((_D, _C), lambda i: (0, 0)),
            pl.BlockSpec((1, _C), lambda i: (0, 0)),
            pl.BlockSpec(memory_space=pltpu.HBM),
            pl.BlockSpec((128,), lambda i: (0,)),
        ],
        out_specs=pl.BlockSpec((_G, _C), lambda i: (0, 0)),
        out_shape=jax.ShapeDtypeStruct((_G, _C), jnp.float32),
        scratch_shapes=[
            pltpu.VMEM((_C, _NC * _G), jnp.float32),
            pltpu.VMEM((_NC * _G, _BLK), jnp.float32),
        ],
    )(x_pad, W, b_row, rp, cnt)


def kernel(x, edge_index, batch, W, b):
    rp_flat, cnt = _sc_call(edge_index, batch)
    rp = rp_flat.reshape(_NC * _G, _NP)
    x_pad = jnp.pad(x, ((0, _NP - _N), (0, 0)))
    return _tc_call(x_pad, W, b.reshape(1, _C), rp, cnt)


# speedup vs baseline: 1.2289x; 1.2289x over previous
import jax
import jax.numpy as jnp
from jax import lax
from jax.experimental import pallas as pl
from jax.experimental.pallas import tpu as pltpu
from jax.experimental.pallas import tpu_sc as plsc

_N = 10000
_E = 320000
_D = 128
_C = 40
_G = 64

_NC = 2
_NS = 16
_NW = _NC * _NS
_L = 16

_NP = 10240
_SLC = _NP // _NS
_EV = _E // _NS
_EW = _E // _NW
_GN = _G * _NP
_RSP = _GN // _NS
_K = 2000
_DK = 4000
_ZCH = 10240


def _rsqrt16(deg):
    magic = jnp.full((_L,), 0x5F3759DF, jnp.int32)
    half = deg * jnp.full((_L,), 0.5, jnp.float32)
    three_half = jnp.full((_L,), 1.5, jnp.float32)
    y = plsc.bitcast(magic - (plsc.bitcast(deg, jnp.int32) >> 1), jnp.float32)
    for _ in range(3):
        y = y * (three_half - half * y * y)
    return y


def _sc_body(ei_hbm, batch_hbm, r_hbm, cnt_hbm,
             edge_v, src_v, batch_v, dis_v, disl_v, z_v, ones_v,
             didx0, didx1, rval0, rval1, ridx0, ridx1, val_s, idx_s,
             s_src, s_dst, s_batch, s_dl0, s_dl1, s_ds0, s_ds1,
             s_rs0, s_rs1, s_o0, s_o1,
             sh_deg, sh_dis, sh_r):
    cid = lax.axis_index("c")
    sid = lax.axis_index("s")
    wid = cid * _NS + sid

    zeros16 = jnp.zeros((_L,), jnp.float32)
    ones16 = jnp.full((_L,), 1.0, jnp.float32)

    didx = (didx0, didx1)
    s_dl = (s_dl0, s_dl1)
    s_ds = (s_ds0, s_ds1)
    rval = (rval0, rval1)
    ridx = (ridx0, ridx1)
    s_rs = (s_rs0, s_rs1)
    outb = (z_v, dis_v)
    s_o = (s_o0, s_o1)
    n_deg = _EV // _DK
    n_r = _EW // _K

    c_src = pltpu.async_copy(ei_hbm.at[pl.ds(wid * _EW, _EW)], src_v, s_src)
    c_dst = pltpu.async_copy(ei_hbm.at[pl.ds(_E + wid * _EW, _EW)],
                             edge_v, s_dst)
    c_bat = pltpu.async_copy(batch_hbm, batch_v.at[pl.ds(0, _N)], s_batch)
    dload = [None] * n_deg
    dload[0] = pltpu.async_copy(
        ei_hbm.at[pl.ds(_E + sid * _EV, _DK)], didx[0], s_dl[0])

    def _z2(i, _):
        z_v[pl.ds(i * _L, _L)] = zeros16
        return ()
    lax.fori_loop(0, _ZCH // _L, _z2, ())

    def _o(i, _):
        ones_v[pl.ds(i * _L, _L)] = ones16
        return ()
    lax.fori_loop(0, _DK // _L, _o, ())

    pltpu.sync_copy(z_v.at[pl.ds(0, _SLC)], sh_deg.at[pl.ds(sid * _SLC, _SLC)])
    for j in range(_RSP // _ZCH):
        pltpu.sync_copy(z_v, sh_r.at[pl.ds(sid * _RSP + j * _ZCH, _ZCH)])

    @pl.when(sid == 0)
    def _():
        pltpu.sync_copy(z_v.at[pl.ds(0, 128)], sh_r.at[pl.ds(_GN, 128)])
    plsc.subcore_barrier()

    dscat = [None] * n_deg
    for ci in range(n_deg):
        b = ci % 2
        dload[ci].wait()
        if ci + 1 < n_deg:
            if ci >= 1:
                dscat[ci - 1].wait()
            dload[ci + 1] = pltpu.async_copy(
                ei_hbm.at[pl.ds(_E + sid * _EV + (ci + 1) * _DK, _DK)],
                didx[1 - b], s_dl[1 - b])
        dscat[ci] = pltpu.async_copy(ones_v, sh_deg.at[didx[b]], s_ds[b],
                                     add=True)
    dscat[n_deg - 2].wait()
    dscat[n_deg - 1].wait()
    plsc.subcore_barrier()

    pltpu.sync_copy(sh_deg.at[pl.ds(sid * _SLC, _SLC)], disl_v)

    def _red(k, _):
        acc = disl_v[pl.ds(k * _L, _L)]
        disv = _rsqrt16(acc + ones16)
        disl_v[pl.ds(k * _L, _L)] = disv
        return ()
    lax.fori_loop(0, _SLC // _L, _red, ())

    pltpu.sync_copy(disl_v, sh_dis.at[pl.ds(sid * _SLC, _SLC)])
    plsc.subcore_barrier()

    pltpu.sync_copy(sh_dis, dis_v)
    c_src.wait()
    c_dst.wait()
    c_bat.wait()

    np16 = jnp.full((_L,), _NP, jnp.int32)

    @pl.when(cid == 0)
    def _():
        nlim16 = jnp.full((_L,), _N, jnp.int32)
        gn16 = jnp.full((_L,), _GN, jnp.int32)
        zeros16i = jnp.zeros((_L,), jnp.int32)

        def _selfloop(k, _):
            off = sid * _SLC + k * _L
            n16 = lax.iota(jnp.int32, _L) + jnp.full((_L,), 1, jnp.int32) * off
            dn = dis_v[pl.ds(off, _L)]
            gb = batch_v[pl.ds(off, _L)]
            valid = n16 < nlim16
            idx_s[pl.ds(2 * k * _L, _L)] = jnp.where(
                valid, gb * np16 + n16, zeros16i)
            val_s[pl.ds(2 * k * _L, _L)] = jnp.where(valid, dn * dn, zeros16)
            idx_s[pl.ds((2 * k + 1) * _L, _L)] = jnp.where(
                valid, gn16 + gb, zeros16i)
            val_s[pl.ds((2 * k + 1) * _L, _L)] = jnp.where(
                valid, ones16, zeros16)
            return ()
        lax.fori_loop(0, _SLC // _L, _selfloop, ())
        pltpu.sync_copy(val_s, sh_r.at[idx_s], add=True)

    rscat = [None] * n_r
    for ci in range(n_r):
        b = ci % 2
        if ci >= 2:
            rscat[ci - 2].wait()

        def _inner(i, _, ci=ci, b=b):
            off = ci * _K + i * _L
            s16 = src_v[pl.ds(off, _L)]
            d16 = edge_v[pl.ds(off, _L)]
            vs = plsc.load_gather(dis_v, [s16])
            vd = plsc.load_gather(dis_v, [d16])
            gb = plsc.load_gather(batch_v, [d16])
            ridx[b][pl.ds(i * _L, _L)] = gb * np16 + s16
            rval[b][pl.ds(i * _L, _L)] = vs * vd
            return ()
        lax.fori_loop(0, _K // _L, _inner, ())
        rscat[ci] = pltpu.async_copy(rval[b], sh_r.at[ridx[b]], s_rs[b],
                                     add=True)
    rscat[n_r - 2].wait()
    rscat[n_r - 1].wait()
    plsc.subcore_barrier()

    ostore = [None, None]
    for j in range(_RSP // _ZCH):
        b = j % 2
        if ostore[b] is not None:
            ostore[b].wait()
        pltpu.sync_copy(sh_r.at[pl.ds(sid * _RSP + j * _ZCH, _ZCH)], outb[b])
        ostore[b] = pltpu.async_copy(
            outb[b],
            r_hbm.at[pl.ds(cid * _GN + sid * _RSP + j * _ZCH, _ZCH)], s_o[b])
    ostore[0].wait()
    ostore[1].wait()

    @pl.when((cid == 0) & (sid == 0))
    def _():
        pltpu.sync_copy(sh_r.at[pl.ds(_GN, 128)], disl_v.at[pl.ds(0, 128)])
        pltpu.sync_copy(disl_v.at[pl.ds(0, 128)], cnt_hbm)


def _sc_call(ei_flat, batch):
    mesh = plsc.VectorSubcoreMesh(core_axis_name="c", subcore_axis_name="s",
                                  num_cores=_NC, num_subcores=_NS)
    return pl.kernel(
        _sc_body,
        out_type=(jax.ShapeDtypeStruct((_NC * _GN,), jnp.float32),
                  jax.ShapeDtypeStruct((128,), jnp.float32)),
        mesh=mesh,
        compiler_params=pltpu.CompilerParams(needs_layout_passes=False),
        scratch_types=[
            pltpu.VMEM((_EW,), jnp.int32),
            pltpu.VMEM((_EW,), jnp.int32),
            pltpu.VMEM((_NP,), jnp.int32),
            pltpu.VMEM((_NP,), jnp.float32),
            pltpu.VMEM((_SLC,), jnp.float32),
            pltpu.VMEM((_ZCH,), jnp.float32),
            pltpu.VMEM((_DK,), jnp.float32),
            pltpu.VMEM((_DK,), jnp.int32),
            pltpu.VMEM((_DK,), jnp.int32),
            pltpu.VMEM((_K,), jnp.float32),
            pltpu.VMEM((_K,), jnp.float32),
            pltpu.VMEM((_K,), jnp.int32),
            pltpu.VMEM((_K,), jnp.int32),
            pltpu.VMEM((2 * _SLC,), jnp.float32),
            pltpu.VMEM((2 * _SLC,), jnp.int32),
            pltpu.SemaphoreType.DMA,
            pltpu.SemaphoreType.DMA,
            pltpu.SemaphoreType.DMA,
            pltpu.SemaphoreType.DMA,
            pltpu.SemaphoreType.DMA,
            pltpu.SemaphoreType.DMA,
            pltpu.SemaphoreType.DMA,
            pltpu.SemaphoreType.DMA,
            pltpu.SemaphoreType.DMA,
            pltpu.SemaphoreType.DMA,
            pltpu.SemaphoreType.DMA,
            pltpu.VMEM_SHARED((_NP,), jnp.float32),
            pltpu.VMEM_SHARED((_NP,), jnp.float32),
            pltpu.VMEM_SHARED((_GN + 128,), jnp.float32),
        ],
    )(ei_flat, batch)


_BLK = 2048
_NBLK = _NP // _BLK
_QB = _BLK // _D


def _tc_body(x_ref, w_ref, b_ref, rp_ref, cnt_ref, out_ref, acc_ref):
    i = pl.program_id(0)

    @pl.when(i == 0)
    def _():
        acc_ref[...] = jnp.zeros((_C, _NC * _G), jnp.float32)

    dn = (((0,), (1,)), ((), ()))
    w = w_ref[...]
    acc = jnp.zeros((_C, _NC * _G), jnp.float32)
    for q in range(_QB):
        hq = jnp.dot(x_ref[q], w, preferred_element_type=jnp.float32)
        acc = acc + lax.dot_general(hq, rp_ref[:, q, :], dn,
                                    preferred_element_type=jnp.float32)
    acc_ref[...] += acc

    @pl.when(i == _NBLK - 1)
    def _():
        accf = acc_ref[...]
        a64 = accf[:, :_G] + accf[:, _G:]
        cnt = cnt_ref[...][: _G].reshape(1, _G)
        pooled_t = a64 / jnp.maximum(cnt, 1.0)
        pooled_t = pooled_t + b_ref[...].T * jnp.where(cnt > 0.0, 1.0, 0.0)
        out_ref[...] = pooled_t.T


def _tc_call(x3, W, b_row, rp3, cnt):
    return pl.pallas_call(
        _tc_body,
        grid=(_NBLK,),
        in_specs=[
            pl.BlockSpec((_QB, _D, _D), lambda i: (i, 0, 0)),
            pl.BlockSpec((_D, _C), lambda i: (0, 0)),
            pl.BlockSpec((1, _C), lambda i: (0, 0)),
            pl.BlockSpec((_NC * _G, _QB, _D), lambda i: (0, i, 0)),
            pl.BlockSpec((128,), lambda i: (0,)),
        ],
        out_specs=pl.BlockSpec((_G, _C), lambda i: (0, 0)),
        out_shape=jax.ShapeDtypeStruct((_G, _C), jnp.float32),
        scratch_shapes=[
            pltpu.VMEM((_C, _NC * _G), jnp.float32),
        ],
    )(x3, W, b_row, rp3, cnt)


def kernel(x, edge_index, batch, W, b):
    ei_flat = edge_index.reshape(2 * _E)
    rp_flat, cnt = _sc_call(ei_flat, batch)
    rp3 = rp_flat.reshape(_NC * _G, _NP // _D, _D)
    x3 = jnp.pad(x, ((0, _NP - _N), (0, 0))).reshape(_NP // _D, _D, _D)
    return _tc_call(x3, W, b.reshape(1, _C), rp3, cnt)


# speedup vs baseline: 1.3108x; 1.0666x over previous
import jax
import jax.numpy as jnp
from jax import lax
from jax.experimental import pallas as pl
from jax.experimental.pallas import tpu as pltpu
from jax.experimental.pallas import tpu_sc as plsc

_N = 10000
_E = 320000
_D = 128
_C = 40
_G = 64

_NC = 2
_NS = 16
_NW = _NC * _NS
_L = 16

_NP = 10240
_SLC = _NP // _NS
_EV = _E // _NS
_EW = _E // _NW
_GN = _G * _NP
_RSP = _GN // _NS
_K = 2000
_DK = 4000
_ZCH = 10240


def _rsqrt16(deg):
    magic = jnp.full((_L,), 0x5F3759DF, jnp.int32)
    half = deg * jnp.full((_L,), 0.5, jnp.float32)
    three_half = jnp.full((_L,), 1.5, jnp.float32)
    y = plsc.bitcast(magic - (plsc.bitcast(deg, jnp.int32) >> 1), jnp.float32)
    for _ in range(3):
        y = y * (three_half - half * y * y)
    return y


def _sc_body(ei_hbm, batch_hbm, r_hbm, cnt_hbm,
             edge_v, src_v, batch_v, dis_v, disl_v, z_v, ones_v,
             didx0, didx1, rval0, rval1, ridx0, ridx1, val_s, idx_s,
             s_src, s_dst, s_batch, s_dl0, s_dl1, s_ds0, s_ds1,
             s_rs0, s_rs1, s_o0, s_o1,
             sh_deg, sh_dis, sh_r):
    cid = lax.axis_index("c")
    sid = lax.axis_index("s")
    wid = cid * _NS + sid

    zeros16 = jnp.zeros((_L,), jnp.float32)
    ones16 = jnp.full((_L,), 1.0, jnp.float32)

    didx = (didx0, didx1)
    s_dl = (s_dl0, s_dl1)
    s_ds = (s_ds0, s_ds1)
    rval = (rval0, rval1)
    ridx = (ridx0, ridx1)
    s_rs = (s_rs0, s_rs1)
    outb = (z_v, dis_v)
    s_o = (s_o0, s_o1)
    n_deg = _EV // _DK
    n_r = _EW // _K

    c_src = pltpu.async_copy(ei_hbm.at[pl.ds(wid * _EW, _EW)], src_v, s_src)
    c_dst = pltpu.async_copy(ei_hbm.at[pl.ds(_E + wid * _EW, _EW)],
                             edge_v, s_dst)
    c_bat = pltpu.async_copy(batch_hbm, batch_v.at[pl.ds(0, _N)], s_batch)
    dload = [None] * n_deg
    dload[0] = pltpu.async_copy(
        ei_hbm.at[pl.ds(_E + sid * _EV, _DK)], didx[0], s_dl[0])

    def _z2(i, _):
        z_v[pl.ds(i * _L, _L)] = zeros16
        return ()
    lax.fori_loop(0, _ZCH // _L, _z2, ())

    def _o(i, _):
        ones_v[pl.ds(i * _L, _L)] = ones16
        return ()
    lax.fori_loop(0, _DK // _L, _o, ())

    pltpu.sync_copy(z_v.at[pl.ds(0, _SLC)], sh_deg.at[pl.ds(sid * _SLC, _SLC)])
    for j in range(_RSP // _ZCH):
        pltpu.sync_copy(z_v, sh_r.at[pl.ds(sid * _RSP + j * _ZCH, _ZCH)])

    @pl.when(sid == 0)
    def _():
        pltpu.sync_copy(z_v.at[pl.ds(0, 128)], sh_r.at[pl.ds(_GN, 128)])
    plsc.subcore_barrier()

    dscat = [None] * n_deg
    for ci in range(n_deg):
        b = ci % 2
        dload[ci].wait()
        if ci + 1 < n_deg:
            if ci >= 1:
                dscat[ci - 1].wait()
            dload[ci + 1] = pltpu.async_copy(
                ei_hbm.at[pl.ds(_E + sid * _EV + (ci + 1) * _DK, _DK)],
                didx[1 - b], s_dl[1 - b])
        dscat[ci] = pltpu.async_copy(ones_v, sh_deg.at[didx[b]], s_ds[b],
                                     add=True)
    dscat[n_deg - 2].wait()
    dscat[n_deg - 1].wait()
    plsc.subcore_barrier()

    pltpu.sync_copy(sh_deg.at[pl.ds(sid * _SLC, _SLC)], disl_v)

    def _red(k, _):
        acc = disl_v[pl.ds(k * _L, _L)]
        disv = _rsqrt16(acc + ones16)
        disl_v[pl.ds(k * _L, _L)] = disv
        return ()
    lax.fori_loop(0, _SLC // _L, _red, ())

    pltpu.sync_copy(disl_v, sh_dis.at[pl.ds(sid * _SLC, _SLC)])
    plsc.subcore_barrier()

    pltpu.sync_copy(sh_dis, dis_v)
    c_src.wait()
    c_dst.wait()
    c_bat.wait()

    np16 = jnp.full((_L,), _NP, jnp.int32)

    @pl.when(cid == 0)
    def _():
        nlim16 = jnp.full((_L,), _N, jnp.int32)
        gn16 = jnp.full((_L,), _GN, jnp.int32)
        zeros16i = jnp.zeros((_L,), jnp.int32)

        def _selfloop(k, _):
            off = sid * _SLC + k * _L
            n16 = lax.iota(jnp.int32, _L) + jnp.full((_L,), 1, jnp.int32) * off
            dn = dis_v[pl.ds(off, _L)]
            gb = batch_v[pl.ds(off, _L)]
            valid = n16 < nlim16
            idx_s[pl.ds(2 * k * _L, _L)] = jnp.where(
                valid, gb * np16 + n16, zeros16i)
            val_s[pl.ds(2 * k * _L, _L)] = jnp.where(valid, dn * dn, zeros16)
            idx_s[pl.ds((2 * k + 1) * _L, _L)] = jnp.where(
                valid, gn16 + gb, zeros16i)
            val_s[pl.ds((2 * k + 1) * _L, _L)] = jnp.where(
                valid, ones16, zeros16)
            return ()
        lax.fori_loop(0, _SLC // _L, _selfloop, ())
        pltpu.sync_copy(val_s, sh_r.at[idx_s], add=True)

    rscat = [None] * n_r
    for ci in range(n_r):
        b = ci % 2
        if ci >= 2:
            rscat[ci - 2].wait()

        def _inner(i, _, ci=ci, b=b):
            off = ci * _K + i * _L
            s16 = src_v[pl.ds(off, _L)]
            d16 = edge_v[pl.ds(off, _L)]
            vs = plsc.load_gather(dis_v, [s16])
            vd = plsc.load_gather(dis_v, [d16])
            gb = plsc.load_gather(batch_v, [d16])
            ridx[b][pl.ds(i * _L, _L)] = gb * np16 + s16
            rval[b][pl.ds(i * _L, _L)] = vs * vd
            return ()
        lax.fori_loop(0, _K // _L, _inner, ())
        rscat[ci] = pltpu.async_copy(rval[b], sh_r.at[ridx[b]], s_rs[b],
                                     add=True)
    rscat[n_r - 2].wait()
    rscat[n_r - 1].wait()
    plsc.subcore_barrier()

    ostore = [None, None]
    for j in range(_RSP // _ZCH):
        b = j % 2
        if ostore[b] is not None:
            ostore[b].wait()
        pltpu.sync_copy(sh_r.at[pl.ds(sid * _RSP + j * _ZCH, _ZCH)], outb[b])
        ostore[b] = pltpu.async_copy(
            outb[b],
            r_hbm.at[pl.ds(cid * _GN + sid * _RSP + j * _ZCH, _ZCH)], s_o[b])
    ostore[0].wait()
    ostore[1].wait()

    @pl.when((cid == 0) & (sid == 0))
    def _():
        pltpu.sync_copy(sh_r.at[pl.ds(_GN, 128)], disl_v.at[pl.ds(0, 128)])
        pltpu.sync_copy(disl_v.at[pl.ds(0, 128)], cnt_hbm)


def _sc_call(ei_flat, batch):
    mesh = plsc.VectorSubcoreMesh(core_axis_name="c", subcore_axis_name="s",
                                  num_cores=_NC, num_subcores=_NS)
    return pl.kernel(
        _sc_body,
        out_type=(jax.ShapeDtypeStruct((_NC * _GN,), jnp.float32),
                  jax.ShapeDtypeStruct((128,), jnp.float32)),
        mesh=mesh,
        compiler_params=pltpu.CompilerParams(needs_layout_passes=False),
        scratch_types=[
            pltpu.VMEM((_EW,), jnp.int32),
            pltpu.VMEM((_EW,), jnp.int32),
            pltpu.VMEM((_NP,), jnp.int32),
            pltpu.VMEM((_NP,), jnp.float32),
            pltpu.VMEM((_SLC,), jnp.float32),
            pltpu.VMEM((_ZCH,), jnp.float32),
            pltpu.VMEM((_DK,), jnp.float32),
            pltpu.VMEM((_DK,), jnp.int32),
            pltpu.VMEM((_DK,), jnp.int32),
            pltpu.VMEM((_K,), jnp.float32),
            pltpu.VMEM((_K,), jnp.float32),
            pltpu.VMEM((_K,), jnp.int32),
            pltpu.VMEM((_K,), jnp.int32),
            pltpu.VMEM((2 * _SLC,), jnp.float32),
            pltpu.VMEM((2 * _SLC,), jnp.int32),
            pltpu.SemaphoreType.DMA,
            pltpu.SemaphoreType.DMA,
            pltpu.SemaphoreType.DMA,
            pltpu.SemaphoreType.DMA,
            pltpu.SemaphoreType.DMA,
            pltpu.SemaphoreType.DMA,
            pltpu.SemaphoreType.DMA,
            pltpu.SemaphoreType.DMA,
            pltpu.SemaphoreType.DMA,
            pltpu.SemaphoreType.DMA,
            pltpu.SemaphoreType.DMA,
            pltpu.VMEM_SHARED((_NP,), jnp.float32),
            pltpu.VMEM_SHARED((_NP,), jnp.float32),
            pltpu.VMEM_SHARED((_GN + 128,), jnp.float32),
        ],
    )(ei_flat, batch)


_BLK = 2048
_NBLK = _NP // _BLK
_QB = _BLK // _D
_XR = 256
_XQ = _BLK // _XR


def _tc_body(x_ref, w_ref, b_ref, rp_ref, cnt_ref, out_ref, acc_ref):
    i = pl.program_id(0)

    @pl.when(i == 0)
    def _():
        acc_ref[...] = jnp.zeros((_C, _NC * _G), jnp.float32)

    w = w_ref[...]
    hs = [jnp.dot(x_ref[j], w, preferred_element_type=jnp.float32)
          for j in range(_XQ)]
    hh = jnp.concatenate(hs, axis=0)
    rr = jnp.concatenate([rp_ref[:, q, :] for q in range(_QB)],
                         axis=1)
    acc_ref[...] += lax.dot_general(hh, rr, (((0,), (1,)), ((), ())),
                                    preferred_element_type=jnp.float32)

    @pl.when(i == _NBLK - 1)
    def _():
        accf = acc_ref[...]
        a64 = accf[:, :_G] + accf[:, _G:]
        cnt = cnt_ref[...][: _G].reshape(1, _G)
        pooled_t = a64 / jnp.maximum(cnt, 1.0)
        pooled_t = pooled_t + b_ref[...].T * jnp.where(cnt > 0.0, 1.0, 0.0)
        out_ref[...] = pooled_t.T


def _tc_call(x3, W, b_row, rp3, cnt):
    return pl.pallas_call(
        _tc_body,
        grid=(_NBLK,),
        in_specs=[
            pl.BlockSpec((_XQ, _XR, _D), lambda i: (i, 0, 0)),
            pl.BlockSpec((_D, _C), lambda i: (0, 0)),
            pl.BlockSpec((1, _C), lambda i: (0, 0)),
            pl.BlockSpec((_NC * _G, _QB, _D), lambda i: (0, i, 0)),
            pl.BlockSpec((128,), lambda i: (0,)),
        ],
        out_specs=pl.BlockSpec((_G, _C), lambda i: (0, 0)),
        out_shape=jax.ShapeDtypeStruct((_G, _C), jnp.float32),
        scratch_shapes=[
            pltpu.VMEM((_C, _NC * _G), jnp.float32),
        ],
    )(x3, W, b_row, rp3, cnt)


def kernel(x, edge_index, batch, W, b):
    ei_flat = edge_index.reshape(2 * _E)
    rp_flat, cnt = _sc_call(ei_flat, batch)
    rp3 = rp_flat.reshape(_NC * _G, _NP // _D, _D)
    x3 = jnp.pad(x, ((0, _NP - _N), (0, 0))).reshape(_NP // _XR, _XR, _D)
    return _tc_call(x3, W, b.reshape(1, _C), rp3, cnt)
